# baseline (speedup 1.0000x reference)
import jax
import jax.numpy as jnp
from jax.experimental import pallas as pl
from jax.experimental.pallas import tpu as pltpu

NUM_EMBEDDINGS = 1024
EMBEDDING_DIM = 32
BETA = 0.25

_B, _C, _D, _H, _W = 4, 32, 16, 32, 32
_SPATIAL = _D * _H * _W
_TN = 4096
_NBLK = _SPATIAL // _TN


def _vq_kernel(z_ref, e_ref, zq_ref, idx_ref, acc_ref, e2_ref, esq_ref):
    b = pl.program_id(0)
    j = pl.program_id(1)
    first = jnp.logical_and(b == 0, j == 0)

    @pl.when(first)
    def _init():
        e0 = e_ref[...]
        e2_ref[...] = -(e0 + e0)
        esq_ref[...] = jnp.sum(e0 * e0, axis=1, keepdims=True)

    zb = z_ref[0].reshape(_C, _TN)
    e = e_ref[...]

    m2 = jax.lax.dot_general(e2_ref[...], zb, (((1,), (0,)), ((), ())),
                             preferred_element_type=jnp.float32)
    z_sq = jnp.sum(zb * zb, axis=0, keepdims=True)
    dist = (z_sq + esq_ref[...]) + m2

    iota_f = jax.lax.broadcasted_iota(
        jnp.int32, (NUM_EMBEDDINGS, _TN), 0).astype(jnp.float32)
    minval = jnp.min(dist, axis=0)
    cand = jnp.where(dist == minval[None, :], iota_f, float(NUM_EMBEDDINGS))
    idx_f = jnp.min(cand, axis=0)

    onehot = (iota_f == idx_f[None, :]).astype(jnp.float32)
    zq = jax.lax.dot_general(e, onehot, (((0,), (0,)), ((), ())),
                             preferred_element_type=jnp.float32)

    zq_ref[0] = (zb + (zq - zb)).reshape(_C, _TN // 32, 32)
    idx_ref[0] = idx_f.astype(jnp.int32).reshape(_TN // 128, 128)

    part = jnp.sum((zq - zb) ** 2).reshape(1, 1)
    acc_ref[...] = jnp.where(first, part, acc_ref[...] + part)


@jax.jit
def kernel(z, embedding):
    z3 = z.reshape(_B, _C, _SPATIAL // 32, 32)
    zq3, idx3, acc = pl.pallas_call(
        _vq_kernel,
        grid=(_B, _NBLK),
        in_specs=[
            pl.BlockSpec((1, _C, _TN // 32, 32), lambda b, j: (b, 0, j, 0)),
            pl.BlockSpec((NUM_EMBEDDINGS, EMBEDDING_DIM), lambda b, j: (0, 0)),
        ],
        out_specs=[
            pl.BlockSpec((1, _C, _TN // 32, 32), lambda b, j: (b, 0, j, 0)),
            pl.BlockSpec((1, _TN // 128, 128), lambda b, j: (b * _NBLK + j, 0, 0)),
            pl.BlockSpec((1, 1), lambda b, j: (0, 0)),
        ],
        out_shape=[
            jax.ShapeDtypeStruct((_B, _C, _SPATIAL // 32, 32), jnp.float32),
            jax.ShapeDtypeStruct((_B * _NBLK, _TN // 128, 128), jnp.int32),
            jax.ShapeDtypeStruct((1, 1), jnp.float32),
        ],
        scratch_shapes=[
            pltpu.VMEM((NUM_EMBEDDINGS, EMBEDDING_DIM), jnp.float32),
            pltpu.VMEM((NUM_EMBEDDINGS, 1), jnp.float32),
        ],
    )(z3, embedding)

    z_q = zq3.reshape(_B, _C, _D, _H, _W)
    encoding_indices = idx3.reshape(-1)
    loss = (1.0 + BETA) * acc[0, 0] / (_B * _SPATIAL * _C)
    return (z_q, loss, encoding_indices)


# speedup vs baseline: 1.0554x; 1.0554x over previous
import jax
import jax.numpy as jnp
from jax.experimental import pallas as pl
from jax.experimental.pallas import tpu as pltpu

NUM_EMBEDDINGS = 1024
EMBEDDING_DIM = 32
BETA = 0.25

_B, _C, _D, _H, _W = 4, 32, 16, 32, 32
_SPATIAL = _D * _H * _W
_TN = 4096
_NBLK = _SPATIAL // _TN


_NG = 32
_GS = NUM_EMBEDDINGS // _NG


def _vq_kernel(z_ref, e_ref, zq_ref, idx_ref, acc_ref, e2_ref, esq_ref, w_ref):
    b = pl.program_id(0)
    j = pl.program_id(1)
    first = jnp.logical_and(b == 0, j == 0)

    @pl.when(first)
    def _init():
        e0 = e_ref[...]
        e2_ref[...] = -(e0 + e0)
        esq_ref[...] = jnp.sum(e0 * e0, axis=1, keepdims=True)
        g_io = jax.lax.broadcasted_iota(jnp.int32, (_NG, NUM_EMBEDDINGS), 0)
        c_io = jax.lax.broadcasted_iota(jnp.int32, (_NG, NUM_EMBEDDINGS), 1)
        local = c_io - g_io * _GS
        inb = jnp.logical_and(local >= 0, local < _GS)
        wf = jax.lax.bitcast_convert_type(
            jax.lax.shift_left(127 - local, 23), jnp.float32)
        w_ref[...] = jnp.where(inb, wf, 0.0)

    zb = z_ref[0].reshape(_C, _TN)
    e = e_ref[...]

    m2 = jax.lax.dot_general(e2_ref[...], zb, (((1,), (0,)), ((), ())),
                             preferred_element_type=jnp.float32)
    z_sq = jnp.sum(zb * zb, axis=0, keepdims=True)
    dist = (z_sq + esq_ref[...]) + m2

    minval = jnp.min(dist, axis=0)
    maskf = (dist == minval[None, :]).astype(jnp.float32)
    s = jax.lax.dot_general(w_ref[...], maskf, (((1,), (0,)), ((), ())),
                            preferred_element_type=jnp.float32)
    g_io = jax.lax.broadcasted_iota(jnp.int32, (_NG, _TN), 0)
    gfirst = jnp.min(jnp.where(s > 0.0, g_io, _NG), axis=0)
    sfirst = jnp.max(jnp.where(g_io == gfirst[None, :], s, 0.0), axis=0)
    expo = jax.lax.shift_right_logical(
        jax.lax.bitcast_convert_type(sfirst, jnp.int32), 23)
    idx = gfirst * _GS + (127 - expo)

    iota_i = jax.lax.broadcasted_iota(jnp.int32, (NUM_EMBEDDINGS, _TN), 0)
    onehot = (iota_i == idx[None, :]).astype(jnp.float32)
    zq = jax.lax.dot_general(e, onehot, (((0,), (0,)), ((), ())),
                             preferred_element_type=jnp.float32)

    zq_ref[0] = (zb + (zq - zb)).reshape(_C, _TN // 32, 32)
    idx_ref[0] = idx.reshape(_TN // 128, 128)

    part = jnp.sum((zq - zb) ** 2).reshape(1, 1)
    acc_ref[...] = jnp.where(first, part, acc_ref[...] + part)


@jax.jit
def kernel(z, embedding):
    z3 = z.reshape(_B, _C, _SPATIAL // 32, 32)
    zq3, idx3, acc = pl.pallas_call(
        _vq_kernel,
        grid=(_B, _NBLK),
        in_specs=[
            pl.BlockSpec((1, _C, _TN // 32, 32), lambda b, j: (b, 0, j, 0)),
            pl.BlockSpec((NUM_EMBEDDINGS, EMBEDDING_DIM), lambda b, j: (0, 0)),
        ],
        out_specs=[
            pl.BlockSpec((1, _C, _TN // 32, 32), lambda b, j: (b, 0, j, 0)),
            pl.BlockSpec((1, _TN // 128, 128), lambda b, j: (b * _NBLK + j, 0, 0)),
            pl.BlockSpec((1, 1), lambda b, j: (0, 0)),
        ],
        out_shape=[
            jax.ShapeDtypeStruct((_B, _C, _SPATIAL // 32, 32), jnp.float32),
            jax.ShapeDtypeStruct((_B * _NBLK, _TN // 128, 128), jnp.int32),
            jax.ShapeDtypeStruct((1, 1), jnp.float32),
        ],
        scratch_shapes=[
            pltpu.VMEM((NUM_EMBEDDINGS, EMBEDDING_DIM), jnp.float32),
            pltpu.VMEM((NUM_EMBEDDINGS, 1), jnp.float32),
            pltpu.VMEM((_NG, NUM_EMBEDDINGS), jnp.float32),
        ],
    )(z3, embedding)

    z_q = zq3.reshape(_B, _C, _D, _H, _W)
    encoding_indices = idx3.reshape(-1)
    loss = (1.0 + BETA) * acc[0, 0] / (_B * _SPATIAL * _C)
    return (z_q, loss, encoding_indices)
